# baseline (speedup 1.0000x reference)
import functools

import jax
import jax.numpy as jnp
from jax import lax
from jax.experimental import pallas as pl
from jax.experimental.pallas import tpu as pltpu
from jax.experimental.pallas import tpu_sc as plsc

N_NODES = 10000
IN_CH = 128
OUT_CH = 128
N_EDGES = 320000

NC = 2
NS = 16
NW = NC * NS
EPW = N_EDGES // NW
CHUNK = 80
NCHUNK = EPW // CHUNK
NPAIR = (NCHUNK + 1) // 2
NV = IN_CH // 16
ACC_W = IN_CH + 16
RPT = N_NODES // NS
PERIOD = 6


def _final_body(x_ref, p_ref, wt_ref, wb_ref, b_ref, o_ref):
    p = p_ref[...]
    ps = p[0] + p[1]
    s1 = ps[:, IN_CH:IN_CH + 1]
    cnt = jnp.maximum(ps[:, IN_CH + 1:IN_CH + 2], 1.0)
    y1 = jnp.dot(x_ref[...], wt_ref[...], preferred_element_type=jnp.float32)
    s2 = jnp.dot(ps[:, :IN_CH], wb_ref[...], preferred_element_type=jnp.float32)
    o_ref[...] = (s2 + y1 * s1) / cnt + b_ref[...]


def _edge_body(x_hbm, idx_hbm, zero_hbm, pm_hbm,
               idx0, idx1, idx2, rows0, rows1, sc_buf,
               semi0, semi1, semi2, semg0, semg1, semm, semz,
               acc_m):
    cid = lax.axis_index("c")
    sid = lax.axis_index("s")
    wid = cid * NS + sid
    idx = (idx0, idx1, idx2)
    semi = (semi0, semi1, semi2)
    rows = (rows0, rows1)
    semg = (semg0, semg1)

    def i_start(off, s3):
        pltpu.async_copy(idx_hbm.at[wid, pl.ds(off, 2)], idx[s3], semi[s3])

    def i_wait(off, s3):
        pltpu.make_async_copy(idx_hbm.at[wid, pl.ds(off, 2)], idx[s3],
                              semi[s3]).wait()

    def g_start(s3, p):
        pltpu.async_copy(x_hbm.at[idx[s3].at[p, 1]], rows[p], semg[p])

    def g_wait(s3, p):
        pltpu.make_async_copy(x_hbm.at[idx[s3].at[p, 1]], rows[p],
                              semg[p]).wait()

    lanes = lax.iota(jnp.int32, 16)

    def compute(s3, p):
        rbuf = idx[s3]
        rows_b = rows[p]

        def group_body(g, _):
            eav = plsc.bitcast(rbuf[p, 2, pl.ds(g * 16, 16)], jnp.float32)
            base = g * 16
            for e16 in range(16):
                ea = eav[e16]
                e = base + e16
                for v in range(NV):
                    sc_buf[e, pl.ds(v * 16, 16)] = rows_b[e, pl.ds(v * 16, 16)] * ea
                sc_buf[e, pl.ds(IN_CH, 16)] = jnp.where(
                    lanes == 0, ea,
                    jnp.where(lanes == 1, jnp.float32(1.0), jnp.float32(0.0)))
            return 0
        lax.fori_loop(0, CHUNK // 16, group_body, 0)

    def s_start(s3, p):
        pltpu.async_copy(sc_buf, acc_m.at[idx[s3].at[p, 0]], semm, add=True)

    def s_wait(s3, p):
        pltpu.make_async_copy(sc_buf, acc_m.at[idx[s3].at[p, 0]], semm).wait()

    def step(c, s3, p, has_next, has_prev, pf_off):
        g_wait(s3, p)
        if has_next:
            if p == 1:
                i_wait(c + 1, (s3 + 1) % 3)
                g_start((s3 + 1) % 3, 0)
            else:
                g_start(s3, 1)
        if has_prev:
            if p == 0:
                s_wait((s3 + 2) % 3, 1)
            else:
                s_wait(s3, 0)
        compute(s3, p)
        s_start(s3, p)
        if p == 0 and pf_off is not None:
            i_start(pf_off, (s3 + 2) % 3)

    i_start(0, 0)
    i_start(2, 1)
    pltpu.async_copy(zero_hbm.at[pl.ds(sid * RPT, RPT)],
                     acc_m.at[pl.ds(sid * RPT, RPT)], semz)
    pltpu.make_async_copy(zero_hbm.at[pl.ds(sid * RPT, RPT)],
                          acc_m.at[pl.ds(sid * RPT, RPT)], semz).wait()
    plsc.subcore_barrier()

    i_wait(0, 0)
    g_start(0, 0)
    step(0, 0, 0, True, False, 4)
    step(1, 0, 1, True, True, None)

    def main_body(i, _):
        c0 = i * PERIOD + 2
        for k in range(PERIOD):
            s3 = (1 + k // 2) % 3
            p = k % 2
            step(c0 + k, s3, p, True, True, c0 + k + 4 if p == 0 else None)
        return 0
    lax.fori_loop(0, 20, main_body, 0)

    step(122, 1, 0, True, True, None)
    step(123, 1, 1, True, True, None)
    step(124, 2, 0, False, True, None)
    s_wait(2, 0)
    plsc.subcore_barrier()

    pltpu.async_copy(acc_m.at[pl.ds(sid * RPT, RPT)],
                     pm_hbm.at[cid, pl.ds(sid * RPT, RPT)], semz)
    pltpu.make_async_copy(acc_m.at[pl.ds(sid * RPT, RPT)],
                          pm_hbm.at[cid, pl.ds(sid * RPT, RPT)], semz).wait()


_edge_call = pl.kernel(
    _edge_body,
    out_type=jax.ShapeDtypeStruct((NC, N_NODES, ACC_W), jnp.float32),
    mesh=plsc.VectorSubcoreMesh(core_axis_name="c", subcore_axis_name="s",
                                num_cores=NC, num_subcores=NS),
    compiler_params=pltpu.CompilerParams(use_tc_tiling_on_sc=False,
                                         needs_layout_passes=False),
    scratch_types=[
        pltpu.VMEM((2, 3, CHUNK), jnp.int32),
        pltpu.VMEM((2, 3, CHUNK), jnp.int32),
        pltpu.VMEM((2, 3, CHUNK), jnp.int32),
        pltpu.VMEM((CHUNK, IN_CH), jnp.float32),
        pltpu.VMEM((CHUNK, IN_CH), jnp.float32),
        pltpu.VMEM((CHUNK, ACC_W), jnp.float32),
        pltpu.SemaphoreType.DMA,
        pltpu.SemaphoreType.DMA,
        pltpu.SemaphoreType.DMA,
        pltpu.SemaphoreType.DMA,
        pltpu.SemaphoreType.DMA,
        pltpu.SemaphoreType.DMA,
        pltpu.SemaphoreType.DMA,
        pltpu.VMEM_SHARED((N_NODES, ACC_W), jnp.float32),
    ],
)

FBLK = 2000

_final_call = pl.pallas_call(
    _final_body,
    grid=(N_NODES // FBLK,),
    in_specs=[
        pl.BlockSpec((FBLK, IN_CH), lambda i: (i, 0)),
        pl.BlockSpec((NC, FBLK, ACC_W), lambda i: (0, i, 0)),
        pl.BlockSpec((IN_CH, OUT_CH), lambda i: (0, 0)),
        pl.BlockSpec((IN_CH, OUT_CH), lambda i: (0, 0)),
        pl.BlockSpec((1, OUT_CH), lambda i: (0, 0)),
    ],
    out_specs=pl.BlockSpec((FBLK, OUT_CH), lambda i: (i, 0)),
    out_shape=jax.ShapeDtypeStruct((N_NODES, OUT_CH), jnp.float32),
)


@jax.jit
def kernel(x, edge_index, edge_attr, weight, bias):
    ei = edge_index.astype(jnp.int32).reshape(2, NW, NCHUNK, CHUNK)
    eab = lax.bitcast_convert_type(edge_attr, jnp.int32).reshape(NW, NCHUNK, CHUNK)
    idx3 = jnp.stack([ei[0], ei[1], eab], axis=2)
    pad = jnp.zeros((NW, 2 * NPAIR - NCHUNK, 3, CHUNK), jnp.int32)
    idx3 = jnp.concatenate([idx3, pad], axis=1)
    zeros = jnp.zeros((N_NODES, ACC_W), jnp.float32)
    pm = _edge_call(x, idx3, zeros)
    return _final_call(x, pm, weight[:IN_CH], weight[IN_CH:],
                       bias.reshape(1, OUT_CH))


# speedup vs baseline: 2.1387x; 2.1387x over previous
import functools

import jax
import jax.numpy as jnp
from jax import lax
from jax.experimental import pallas as pl
from jax.experimental.pallas import tpu as pltpu
from jax.experimental.pallas import tpu_sc as plsc

N_NODES = 10000
IN_CH = 128
OUT_CH = 128
N_EDGES = 320000

NC = 2
NS = 16
NW = NC * NS
EPW = N_EDGES // NW
CHUNK = 80
NCHUNK = EPW // CHUNK
NPAIR = (NCHUNK + 1) // 2
NV = IN_CH // 16
RPT = N_NODES // NS
PERIOD = 6


def _final_body(x_ref, z_ref, pe_ref, wt_ref, wb_ref, b_ref, o_ref):
    z = z_ref[...]
    zs = z[0] + z[1]
    pe = pe_ref[...]
    pes = pe[0] + pe[1]
    s1 = pes[:, 0:1]
    cnt = jnp.maximum(pes[:, 1:2], 1.0)
    y1 = jnp.dot(x_ref[...], wt_ref[...], preferred_element_type=jnp.float32)
    s2 = jnp.dot(zs, wb_ref[...], preferred_element_type=jnp.float32)
    o_ref[...] = (s2 + y1 * s1) / cnt + b_ref[...]


def _edge_body(x_hbm, idx_hbm, zm_hbm, ze_hbm, pm_hbm, pe_hbm,
               idx0, idx1, idx2, rows0, rows1, rows2, ex0, ex1,
               semi0, semi1, semi2, semg0, semg1, semg2,
               semm0, semm1, seme0, seme1, semz,
               acc_m, acc_e):
    cid = lax.axis_index("c")
    sid = lax.axis_index("s")
    wid = cid * NS + sid
    idx = (idx0, idx1, idx2)
    semi = (semi0, semi1, semi2)
    rows = (rows0, rows1, rows2)
    semg = (semg0, semg1, semg2)
    ex = (ex0, ex1)
    semm = (semm0, semm1)
    seme = (seme0, seme1)

    def i_start(off, s3):
        pltpu.async_copy(idx_hbm.at[wid, pl.ds(off, 2)], idx[s3], semi[s3])

    def i_wait(off, s3):
        pltpu.make_async_copy(idx_hbm.at[wid, pl.ds(off, 2)], idx[s3],
                              semi[s3]).wait()

    def g_start(s3, p, r3):
        pltpu.async_copy(x_hbm.at[idx[s3].at[p, 1]], rows[r3], semg[r3])

    def g_wait(s3, p, r3):
        pltpu.make_async_copy(x_hbm.at[idx[s3].at[p, 1]], rows[r3],
                              semg[r3]).wait()

    lanes = lax.iota(jnp.int32, 16)

    def compute(s3, p, r3):
        rbuf = idx[s3]
        rows_b = rows[r3]
        ex_b = ex[p]

        def group_body(g, _):
            eav = plsc.bitcast(rbuf[p, 2, pl.ds(g * 16, 16)], jnp.float32)
            base = g * 16
            for e16 in range(16):
                ea = eav[e16]
                e = base + e16
                for v in range(NV):
                    rows_b[e, pl.ds(v * 16, 16)] = rows_b[e, pl.ds(v * 16, 16)] * ea
                ex_b[e, :] = jnp.where(
                    lanes == 0, ea,
                    jnp.where(lanes == 1, jnp.float32(1.0), jnp.float32(0.0)))
            return 0
        lax.fori_loop(0, CHUNK // 16, group_body, 0)

    def s_start(s3, p, r3):
        pltpu.async_copy(rows[r3], acc_m.at[idx[s3].at[p, 0]], semm[p], add=True)
        pltpu.async_copy(ex[p], acc_e.at[idx[s3].at[p, 0]], seme[p], add=True)

    def s_wait(s3, p, r3):
        pltpu.make_async_copy(rows[r3], acc_m.at[idx[s3].at[p, 0]], semm[p]).wait()
        pltpu.make_async_copy(ex[p], acc_e.at[idx[s3].at[p, 0]], seme[p]).wait()

    def step(c, s3, p, r3, has_next, has_prev, pf):
        g_wait(s3, p, r3)
        if has_next:
            if p == 1:
                i_wait(c + 1, (s3 + 1) % 3)
                g_start((s3 + 1) % 3, 0, (r3 + 1) % 3)
            else:
                g_start(s3, 1, (r3 + 1) % 3)
        compute(s3, p, r3)
        s_start(s3, p, r3)
        if has_prev:
            s_wait((s3 + 2) % 3 if p == 0 else s3, 1 - p, (r3 + 2) % 3)
        if p == 0 and pf:
            i_start(c + 4, (s3 + 2) % 3)

    i_start(0, 0)
    i_start(2, 1)
    pltpu.async_copy(zm_hbm.at[pl.ds(sid * RPT, RPT)],
                     acc_m.at[pl.ds(sid * RPT, RPT)], semz)
    pltpu.async_copy(ze_hbm.at[pl.ds(sid * RPT, RPT)],
                     acc_e.at[pl.ds(sid * RPT, RPT)], semz)
    pltpu.make_async_copy(zm_hbm.at[pl.ds(sid * RPT, RPT)],
                          acc_m.at[pl.ds(sid * RPT, RPT)], semz).wait()
    pltpu.make_async_copy(ze_hbm.at[pl.ds(sid * RPT, RPT)],
                          acc_e.at[pl.ds(sid * RPT, RPT)], semz).wait()
    plsc.subcore_barrier()

    i_wait(0, 0)
    g_start(0, 0, 0)
    step(0, 0, 0, 0, True, False, True)
    step(1, 0, 1, 1, True, True, False)

    def main_body(i, _):
        c0 = i * PERIOD + 2
        for k in range(PERIOD):
            s3 = (1 + k // 2) % 3
            p = k % 2
            r3 = (2 + k) % 3
            step(c0 + k, s3, p, r3, True, True, p == 0)
        return 0
    lax.fori_loop(0, 20, main_body, 0)

    step(122, 1, 0, 2, True, True, False)
    step(123, 1, 1, 0, True, True, False)
    step(124, 2, 0, 1, False, True, False)
    s_wait(2, 0, 1)
    plsc.subcore_barrier()

    pltpu.async_copy(acc_m.at[pl.ds(sid * RPT, RPT)],
                     pm_hbm.at[cid, pl.ds(sid * RPT, RPT)], semz)
    pltpu.async_copy(acc_e.at[pl.ds(sid * RPT, RPT)],
                     pe_hbm.at[cid, pl.ds(sid * RPT, RPT)], semz)
    pltpu.make_async_copy(acc_m.at[pl.ds(sid * RPT, RPT)],
                          pm_hbm.at[cid, pl.ds(sid * RPT, RPT)], semz).wait()
    pltpu.make_async_copy(acc_e.at[pl.ds(sid * RPT, RPT)],
                          pe_hbm.at[cid, pl.ds(sid * RPT, RPT)], semz).wait()


_edge_call = pl.kernel(
    _edge_body,
    out_type=[
        jax.ShapeDtypeStruct((NC, N_NODES, IN_CH), jnp.float32),
        jax.ShapeDtypeStruct((NC, N_NODES, 16), jnp.float32),
    ],
    mesh=plsc.VectorSubcoreMesh(core_axis_name="c", subcore_axis_name="s",
                                num_cores=NC, num_subcores=NS),
    compiler_params=pltpu.CompilerParams(use_tc_tiling_on_sc=False,
                                         needs_layout_passes=False),
    scratch_types=[
        pltpu.VMEM((2, 3, CHUNK), jnp.int32),
        pltpu.VMEM((2, 3, CHUNK), jnp.int32),
        pltpu.VMEM((2, 3, CHUNK), jnp.int32),
        pltpu.VMEM((CHUNK, IN_CH), jnp.float32),
        pltpu.VMEM((CHUNK, IN_CH), jnp.float32),
        pltpu.VMEM((CHUNK, IN_CH), jnp.float32),
        pltpu.VMEM((CHUNK, 16), jnp.float32),
        pltpu.VMEM((CHUNK, 16), jnp.float32),
        pltpu.SemaphoreType.DMA,
        pltpu.SemaphoreType.DMA,
        pltpu.SemaphoreType.DMA,
        pltpu.SemaphoreType.DMA,
        pltpu.SemaphoreType.DMA,
        pltpu.SemaphoreType.DMA,
        pltpu.SemaphoreType.DMA,
        pltpu.SemaphoreType.DMA,
        pltpu.SemaphoreType.DMA,
        pltpu.SemaphoreType.DMA,
        pltpu.SemaphoreType.DMA,
        pltpu.VMEM_SHARED((N_NODES, IN_CH), jnp.float32),
        pltpu.VMEM_SHARED((N_NODES, 16), jnp.float32),
    ],
)

FBLK = 2000

_final_call = pl.pallas_call(
    _final_body,
    grid=(N_NODES // FBLK,),
    in_specs=[
        pl.BlockSpec((FBLK, IN_CH), lambda i: (i, 0)),
        pl.BlockSpec((NC, FBLK, IN_CH), lambda i: (0, i, 0)),
        pl.BlockSpec((NC, FBLK, 16), lambda i: (0, i, 0)),
        pl.BlockSpec((IN_CH, OUT_CH), lambda i: (0, 0)),
        pl.BlockSpec((IN_CH, OUT_CH), lambda i: (0, 0)),
        pl.BlockSpec((1, OUT_CH), lambda i: (0, 0)),
    ],
    out_specs=pl.BlockSpec((FBLK, OUT_CH), lambda i: (i, 0)),
    out_shape=jax.ShapeDtypeStruct((N_NODES, OUT_CH), jnp.float32),
)


@jax.jit
def kernel(x, edge_index, edge_attr, weight, bias):
    ei = edge_index.astype(jnp.int32).reshape(2, NW, NCHUNK, CHUNK)
    eab = lax.bitcast_convert_type(edge_attr, jnp.int32).reshape(NW, NCHUNK, CHUNK)
    idx3 = jnp.stack([ei[0], ei[1], eab], axis=2)
    pad = jnp.zeros((NW, 2 * NPAIR - NCHUNK, 3, CHUNK), jnp.int32)
    idx3 = jnp.concatenate([idx3, pad], axis=1)
    zm = jnp.zeros((N_NODES, IN_CH), jnp.float32)
    ze = jnp.zeros((N_NODES, 16), jnp.float32)
    zmo, pe = _edge_call(x, idx3, zm, ze)
    return _final_call(x, zmo, pe, weight[:IN_CH], weight[IN_CH:],
                       bias.reshape(1, OUT_CH))


# speedup vs baseline: 2.8255x; 1.3211x over previous
import functools

import jax
import jax.numpy as jnp
from jax import lax
from jax.experimental import pallas as pl
from jax.experimental.pallas import tpu as pltpu
from jax.experimental.pallas import tpu_sc as plsc

N_NODES = 10000
IN_CH = 128
OUT_CH = 128
N_EDGES = 320000

NC = 2
NS = 16
NW = NC * NS
EPW = N_EDGES // NW
CHUNK = 80
NCHUNK = EPW // CHUNK
NV = IN_CH // 16
NSTRIPE = N_NODES // CHUNK
PERIOD = 6


def _final_body(x_ref, z_ref, pe_ref, wt_ref, wb_ref, b_ref, o_ref):
    z = z_ref[...]
    zs = z[0] + z[1]
    pe = pe_ref[...]
    pes = pe[0] + pe[1]
    s1 = pes[:, 0:1]
    cnt = jnp.maximum(pes[:, 1:2], 1.0)
    y1 = jnp.dot(x_ref[...], wt_ref[...], preferred_element_type=jnp.float32)
    s2 = jnp.dot(zs, wb_ref[...], preferred_element_type=jnp.float32)
    o_ref[...] = (s2 + y1 * s1) / cnt + b_ref[...]


def _edge_body(x_hbm, ei_hbm, ea_hbm, pm_hbm, pe_hbm,
               ir0, ir1, ir2, ic0, ic1, ic2, ev0, ev1, ev2,
               rows0, rows1, rows2, ex0, ex1,
               semi0, semi1, semi2, semg0, semg1, semg2,
               semm0, semm1, seme0, seme1, semz,
               acc_m, acc_e):
    cid = lax.axis_index("c")
    sid = lax.axis_index("s")
    wid = cid * NS + sid
    ir = (ir0, ir1, ir2)
    ic = (ic0, ic1, ic2)
    ev = (ev0, ev1, ev2)
    semi = (semi0, semi1, semi2)
    rows = (rows0, rows1, rows2)
    semg = (semg0, semg1, semg2)
    ex = (ex0, ex1)
    semm = (semm0, semm1)
    seme = (seme0, seme1)

    def i_start(off, s3):
        pltpu.async_copy(ei_hbm.at[0, wid, pl.ds(off, 2)], ir[s3], semi[s3])
        pltpu.async_copy(ei_hbm.at[1, wid, pl.ds(off, 2)], ic[s3], semi[s3])
        pltpu.async_copy(ea_hbm.at[wid, pl.ds(off, 2)], ev[s3], semi[s3])

    def i_wait(off, s3):
        pltpu.make_async_copy(ei_hbm.at[0, wid, pl.ds(off, 2)], ir[s3],
                              semi[s3]).wait()
        pltpu.make_async_copy(ei_hbm.at[1, wid, pl.ds(off, 2)], ic[s3],
                              semi[s3]).wait()
        pltpu.make_async_copy(ea_hbm.at[wid, pl.ds(off, 2)], ev[s3],
                              semi[s3]).wait()

    def g_start(s3, p, r3):
        pltpu.async_copy(x_hbm.at[ic[s3].at[p]], rows[r3], semg[r3])

    def g_wait(s3, p, r3):
        pltpu.make_async_copy(x_hbm.at[ic[s3].at[p]], rows[r3],
                              semg[r3]).wait()

    lanes = lax.iota(jnp.int32, 16)

    def compute(s3, p, r3):
        rows_b = rows[r3]
        ex_b = ex[p]
        evb = ev[s3]

        def group_body(g, _):
            eav = evb[p, pl.ds(g * 16, 16)]
            base = g * 16
            for e16 in range(16):
                ea = eav[e16]
                e = base + e16
                for v in range(NV):
                    rows_b[e, pl.ds(v * 16, 16)] = rows_b[e, pl.ds(v * 16, 16)] * ea
                ex_b[e, :] = jnp.where(
                    lanes == 0, ea,
                    jnp.where(lanes == 1, jnp.float32(1.0), jnp.float32(0.0)))
            return 0
        lax.fori_loop(0, CHUNK // 16, group_body, 0)

    def s_start(s3, p, r3):
        pltpu.async_copy(rows[r3], acc_m.at[ir[s3].at[p]], semm[p], add=True)
        pltpu.async_copy(ex[p], acc_e.at[ir[s3].at[p]], seme[p], add=True)

    def s_wait(s3, p, r3):
        pltpu.make_async_copy(rows[r3], acc_m.at[ir[s3].at[p]], semm[p]).wait()
        pltpu.make_async_copy(ex[p], acc_e.at[ir[s3].at[p]], seme[p]).wait()

    def step(c, s3, p, r3, has_next, has_prev, pf):
        g_wait(s3, p, r3)
        if has_next:
            if p == 1:
                i_wait(c + 1, (s3 + 1) % 3)
                g_start((s3 + 1) % 3, 0, (r3 + 1) % 3)
            else:
                g_start(s3, 1, (r3 + 1) % 3)
        compute(s3, p, r3)
        s_start(s3, p, r3)
        if has_prev:
            s_wait((s3 + 2) % 3 if p == 0 else s3, 1 - p, (r3 + 2) % 3)
        if pf:
            @pl.when(c + 5 < NCHUNK)
            def _():
                i_start(c + 4, (s3 + 2) % 3)

    i_start(0, 0)
    i_start(2, 1)

    def zrow(i, _):
        for v in range(NV):
            rows0[i, pl.ds(v * 16, 16)] = jnp.zeros((16,), jnp.float32)
        ex0[i, :] = jnp.zeros((16,), jnp.float32)
        return 0
    lax.fori_loop(0, CHUNK, zrow, 0)
    NJ = (NSTRIPE + NS - 1) // NS
    for j in range(NJ):
        st = sid + NS * j
        @pl.when(st < NSTRIPE)
        def _():
            pltpu.async_copy(rows0, acc_m.at[pl.ds(st * CHUNK, CHUNK)], semz)
            pltpu.async_copy(ex0, acc_e.at[pl.ds(st * CHUNK, CHUNK)], semz)
    for j in range(NJ):
        st = sid + NS * j
        @pl.when(st < NSTRIPE)
        def _():
            pltpu.make_async_copy(rows0, acc_m.at[pl.ds(st * CHUNK, CHUNK)], semz).wait()
            pltpu.make_async_copy(ex0, acc_e.at[pl.ds(st * CHUNK, CHUNK)], semz).wait()
    plsc.subcore_barrier()

    i_wait(0, 0)
    g_start(0, 0, 0)
    step(0, 0, 0, 0, True, False, True)
    step(1, 0, 1, 1, True, True, False)

    def main_body(i, _):
        c0 = i * PERIOD + 2
        for k in range(PERIOD):
            s3 = (1 + k // 2) % 3
            p = k % 2
            r3 = (2 + k) % 3
            step(c0 + k, s3, p, r3, True, True, p == 0)
        return 0
    lax.fori_loop(0, 20, main_body, 0)

    i_start(123, 2)
    step(122, 1, 0, 2, True, True, False)
    g_wait(1, 1, 0)
    i_wait(123, 2)
    g_start(2, 1, 1)
    compute(1, 1, 0)
    s_start(1, 1, 0)
    s_wait(1, 0, 2)
    g_wait(2, 1, 1)
    s_wait(1, 1, 0)
    compute(2, 1, 1)
    s_start(2, 1, 1)
    s_wait(2, 1, 1)
    plsc.subcore_barrier()

    for j in range(NJ):
        st = sid + NS * j
        @pl.when(st < NSTRIPE)
        def _():
            pltpu.async_copy(acc_m.at[pl.ds(st * CHUNK, CHUNK)],
                             pm_hbm.at[cid, pl.ds(st * CHUNK, CHUNK)], semz)
            pltpu.async_copy(acc_e.at[pl.ds(st * CHUNK, CHUNK)],
                             pe_hbm.at[cid, pl.ds(st * CHUNK, CHUNK)], semz)
    for j in range(NJ):
        st = sid + NS * j
        @pl.when(st < NSTRIPE)
        def _():
            pltpu.make_async_copy(acc_m.at[pl.ds(st * CHUNK, CHUNK)],
                                  pm_hbm.at[cid, pl.ds(st * CHUNK, CHUNK)], semz).wait()
            pltpu.make_async_copy(acc_e.at[pl.ds(st * CHUNK, CHUNK)],
                                  pe_hbm.at[cid, pl.ds(st * CHUNK, CHUNK)], semz).wait()


_edge_call = pl.kernel(
    _edge_body,
    out_type=[
        jax.ShapeDtypeStruct((NC, N_NODES, IN_CH), jnp.float32),
        jax.ShapeDtypeStruct((NC, N_NODES, 16), jnp.float32),
    ],
    mesh=plsc.VectorSubcoreMesh(core_axis_name="c", subcore_axis_name="s",
                                num_cores=NC, num_subcores=NS),
    compiler_params=pltpu.CompilerParams(use_tc_tiling_on_sc=False,
                                         needs_layout_passes=False),
    scratch_types=[
        pltpu.VMEM((2, CHUNK), jnp.int32),
        pltpu.VMEM((2, CHUNK), jnp.int32),
        pltpu.VMEM((2, CHUNK), jnp.int32),
        pltpu.VMEM((2, CHUNK), jnp.int32),
        pltpu.VMEM((2, CHUNK), jnp.int32),
        pltpu.VMEM((2, CHUNK), jnp.int32),
        pltpu.VMEM((2, CHUNK), jnp.float32),
        pltpu.VMEM((2, CHUNK), jnp.float32),
        pltpu.VMEM((2, CHUNK), jnp.float32),
        pltpu.VMEM((CHUNK, IN_CH), jnp.float32),
        pltpu.VMEM((CHUNK, IN_CH), jnp.float32),
        pltpu.VMEM((CHUNK, IN_CH), jnp.float32),
        pltpu.VMEM((CHUNK, 16), jnp.float32),
        pltpu.VMEM((CHUNK, 16), jnp.float32),
        pltpu.SemaphoreType.DMA,
        pltpu.SemaphoreType.DMA,
        pltpu.SemaphoreType.DMA,
        pltpu.SemaphoreType.DMA,
        pltpu.SemaphoreType.DMA,
        pltpu.SemaphoreType.DMA,
        pltpu.SemaphoreType.DMA,
        pltpu.SemaphoreType.DMA,
        pltpu.SemaphoreType.DMA,
        pltpu.SemaphoreType.DMA,
        pltpu.SemaphoreType.DMA,
        pltpu.VMEM_SHARED((N_NODES, IN_CH), jnp.float32),
        pltpu.VMEM_SHARED((N_NODES, 16), jnp.float32),
    ],
)

FBLK = 2000

_final_call = pl.pallas_call(
    _final_body,
    grid=(N_NODES // FBLK,),
    in_specs=[
        pl.BlockSpec((FBLK, IN_CH), lambda i: (i, 0)),
        pl.BlockSpec((NC, FBLK, IN_CH), lambda i: (0, i, 0)),
        pl.BlockSpec((NC, FBLK, 16), lambda i: (0, i, 0)),
        pl.BlockSpec((IN_CH, OUT_CH), lambda i: (0, 0)),
        pl.BlockSpec((IN_CH, OUT_CH), lambda i: (0, 0)),
        pl.BlockSpec((1, OUT_CH), lambda i: (0, 0)),
    ],
    out_specs=pl.BlockSpec((FBLK, OUT_CH), lambda i: (i, 0)),
    out_shape=jax.ShapeDtypeStruct((N_NODES, OUT_CH), jnp.float32),
)


@jax.jit
def kernel(x, edge_index, edge_attr, weight, bias):
    ei = edge_index.astype(jnp.int32).reshape(2, NW, NCHUNK, CHUNK)
    ea = edge_attr.reshape(NW, NCHUNK, CHUNK)
    zm, pe = _edge_call(x, ei, ea)
    return _final_call(x, zm, pe, weight[:IN_CH], weight[IN_CH:],
                       bias.reshape(1, OUT_CH))
